# Initial kernel scaffold; baseline (speedup 1.0000x reference)
import jax
import jax.numpy as jnp
from jax.experimental import pallas as pl


def kernel(h, x, edges, edge_attr, params):
    raise NotImplementedError("write your pallas kernel here")


# speedup vs baseline: 2.2267x; 2.2267x over previous
import functools

import jax
import jax.numpy as jnp
from jax import lax
from jax.experimental import pallas as pl
from jax.experimental.pallas import tpu as pltpu
from jax.experimental.pallas import tpu_sc as plsc

F32 = jnp.float32

NC = 2
NS = 16
NW = NC * NS
CH = 128
NPAD = 10240
RPS = NPAD // NS

@functools.cache
def _mesh():
    return plsc.VectorSubcoreMesh(
        core_axis_name="c", subcore_axis_name="s",
        num_cores=NC, num_subcores=NS,
    )


@functools.cache
def _make_gather2(n_rows, width, n_edges):
    per = n_edges // NW
    assert per * NW == n_edges
    n_full = per // CH
    tail = per - n_full * CH
    assert tail % 8 == 0

    scratch = [
        pltpu.VMEM((CH,), jnp.int32),
        pltpu.VMEM((CH,), jnp.int32),
        pltpu.VMEM((CH, width), F32),
        pltpu.VMEM((CH, width), F32),
        pltpu.SemaphoreType.DMA,
        pltpu.SemaphoreType.DMA,
    ]
    if tail:
        scratch += [
            pltpu.VMEM((tail,), jnp.int32),
            pltpu.VMEM((tail,), jnp.int32),
            pltpu.VMEM((tail, width), F32),
            pltpu.VMEM((tail, width), F32),
        ]

    @functools.partial(
        pl.kernel,
        out_type=(
            jax.ShapeDtypeStruct((n_edges, width), F32),
            jax.ShapeDtypeStruct((n_edges, width), F32),
        ),
        mesh=_mesh(),
        scratch_types=scratch,
    )
    def gather2(ta, tb, ia, ib, oa, ob, iva, ivb, bufa, bufb, sema, semb,
                *tails):
        wid = lax.axis_index("s") * NC + lax.axis_index("c")
        base0 = wid * per

        def chunk(base, iva, ivb, bufa, bufb, n):
            pltpu.sync_copy(ia.at[pl.ds(base, n)], iva)
            pltpu.sync_copy(ib.at[pl.ds(base, n)], ivb)
            ca = pltpu.async_copy(ta.at[iva], bufa, sema)
            cb = pltpu.async_copy(tb.at[ivb], bufb, semb)
            ca.wait()
            cb.wait()
            pltpu.sync_copy(bufa, oa.at[pl.ds(base, n)])
            pltpu.sync_copy(bufb, ob.at[pl.ds(base, n)])

        def body(it, carry):
            chunk(base0 + it * CH, iva, ivb, bufa, bufb, CH)
            return carry

        lax.fori_loop(0, n_full, body, 0)
        if tail:
            tia, tib, tba, tbb = tails
            chunk(base0 + n_full * CH, tia, tib, tba, tbb, tail)

    return gather2


@functools.cache
def _make_segsum(n_edges, width):
    half = width // NC
    per = n_edges // NS
    assert per * NS == n_edges
    n_full = per // CH
    tail = per - n_full * CH
    assert tail % 8 == 0

    scratch = [
        pltpu.VMEM((CH,), jnp.int32),
        pltpu.VMEM((CH, half), F32),
        pltpu.VMEM_SHARED((NPAD, half), F32),
    ]
    if tail:
        scratch += [
            pltpu.VMEM((tail,), jnp.int32),
            pltpu.VMEM((tail, half), F32),
        ]

    @functools.partial(
        pl.kernel,
        out_type=jax.ShapeDtypeStruct((NPAD, width), F32),
        mesh=_mesh(),
        scratch_types=scratch,
    )
    def segsum(vals, rows, zeros, out, idxv, buf, acc, *tails):
        cid = lax.axis_index("c")
        sid = lax.axis_index("s")
        pltpu.sync_copy(
            zeros.at[pl.ds(sid * RPS, RPS), pl.ds(0, half)],
            acc.at[pl.ds(sid * RPS, RPS)],
        )
        plsc.subcore_barrier()

        base0 = sid * per

        def chunk(base, idxv, buf, n):
            pltpu.sync_copy(rows.at[pl.ds(base, n)], idxv)
            pltpu.sync_copy(
                vals.at[pl.ds(base, n), pl.ds(cid * half, half)], buf
            )
            pltpu.sync_copy(buf, acc.at[idxv], add=True)

        def body(it, carry):
            chunk(base0 + it * CH, idxv, buf, CH)
            return carry

        lax.fori_loop(0, n_full, body, 0)
        if tail:
            tidx, tbuf = tails
            chunk(base0 + n_full * CH, tidx, tbuf, tail)

        plsc.subcore_barrier()
        pltpu.sync_copy(
            acc.at[pl.ds(sid * RPS, RPS)],
            out.at[pl.ds(sid * RPS, RPS), pl.ds(cid * half, half)],
        )

    return segsum


@functools.cache
def _make_segsum_part(n_edges):
    width = 128
    per_core = n_edges // NC
    per = per_core // NS
    n_full = per // CH
    tail = per - n_full * CH
    assert tail % 8 == 0

    scratch = [
        pltpu.VMEM((CH,), jnp.int32),
        pltpu.VMEM((CH, width), F32),
        pltpu.VMEM_SHARED((NPAD, width), F32),
    ]
    if tail:
        scratch += [
            pltpu.VMEM((tail,), jnp.int32),
            pltpu.VMEM((tail, width), F32),
        ]

    @functools.partial(
        pl.kernel,
        out_type=jax.ShapeDtypeStruct((NC, NPAD, width), F32),
        mesh=_mesh(),
        scratch_types=scratch,
    )
    def segsum_part(vals, rows, zeros, out, idxv, buf, acc, *tails):
        cid = lax.axis_index("c")
        sid = lax.axis_index("s")
        pltpu.sync_copy(
            zeros.at[pl.ds(sid * RPS, RPS)],
            acc.at[pl.ds(sid * RPS, RPS)],
        )
        plsc.subcore_barrier()

        base0 = cid * per_core + sid * per

        def chunk(base, idxv, buf, n):
            pltpu.sync_copy(rows.at[pl.ds(base, n)], idxv)
            pltpu.sync_copy(vals.at[pl.ds(base, n)], buf)
            pltpu.sync_copy(buf, acc.at[idxv], add=True)

        def body(it, carry):
            chunk(base0 + it * CH, idxv, buf, CH)
            return carry

        lax.fori_loop(0, n_full, body, 0)
        if tail:
            tidx, tbuf = tails
            chunk(base0 + n_full * CH, tidx, tbuf, tail)

        plsc.subcore_barrier()
        pltpu.sync_copy(
            acc.at[pl.ds(sid * RPS, RPS)],
            out.at[cid, pl.ds(sid * RPS, RPS)],
        )

    return segsum_part


def _silu(v):
    return v * jax.nn.sigmoid(v)


def _dot(a, b):
    return jnp.dot(a, b, preferred_element_type=F32)


_BN = 2000
_BE = 1600


def _full(shape):
    return pl.BlockSpec(shape, lambda i: (0,) * len(shape))


def _proj_body(h, wr, wc, hr, hc):
    hv = h[...]
    hr[...] = _dot(hv, wr[...])
    hc[...] = _dot(hv, wc[...])


def _proj(h, wr, wc):
    n, d = h.shape
    return pl.pallas_call(
        _proj_body,
        grid=(n // _BN,),
        in_specs=[
            pl.BlockSpec((_BN, d), lambda i: (i, 0)),
            _full((d, d)),
            _full((d, d)),
        ],
        out_specs=[pl.BlockSpec((_BN, d), lambda i: (i, 0))] * 2,
        out_shape=[jax.ShapeDtypeStruct((n, d), F32)] * 2,
    )(h, wr, wc)


def _diff_body(xr, xc, out):
    out[...] = (xr[...] - xc[...])[:, :16]


def _coord_diff(xr128, xc128):
    e = xr128.shape[0]
    spec = pl.BlockSpec((_BE, 128), lambda i: (i, 0))
    return pl.pallas_call(
        _diff_body,
        grid=(e // _BE,),
        in_specs=[spec, spec],
        out_specs=pl.BlockSpec((_BE, 16), lambda i: (i, 0)),
        out_shape=jax.ShapeDtypeStruct((e, 16), F32),
    )(xr128, xc128)


def _edge_body(gr, gc, diff, ea, wea, wrad, be1, we2, be2, out):
    d = diff[...]
    radial = jnp.sum(d * d, axis=1, keepdims=True)
    pre = (
        gr[...] + gc[...] + _dot(ea[...], wea[...])
        + radial * wrad[...] + be1[...]
    )
    m = _silu(pre)
    out[...] = _silu(_dot(m, we2[...]) + be2[...])


def _edge_last_body(gr, gc, diff, ea, wea, wrad, be1, we2, be2,
                    wc1, bc1, wc2t, out, trans):
    d = diff[...]
    radial = jnp.sum(d * d, axis=1, keepdims=True)
    pre = (
        gr[...] + gc[...] + _dot(ea[...], wea[...])
        + radial * wrad[...] + be1[...]
    )
    m = _silu(pre)
    m2 = _silu(_dot(m, we2[...]) + be2[...])
    out[...] = m2
    c1 = _silu(_dot(m2, wc1[...]) + bc1[...])
    w = jnp.sum(c1 * wc2t[...], axis=1, keepdims=True)
    trans[...] = jnp.concatenate(
        [d * w, jnp.zeros((d.shape[0], 112), F32)], axis=1
    )


def _edge_mlp(gr, gc, diff16, ea, wea, wrad, be1, we2, be2, coord=None):
    e, d = gr.shape
    de = ea.shape[1]
    edge_spec = pl.BlockSpec((_BE, d), lambda i: (i, 0))
    nar_spec = pl.BlockSpec((_BE, 16), lambda i: (i, 0))
    ea_spec = pl.BlockSpec((_BE, de), lambda i: (i, 0))
    in_specs = [
        edge_spec, edge_spec, nar_spec, ea_spec,
        _full((de, d)), _full((1, d)), _full((1, d)),
        _full((d, d)), _full((1, d)),
    ]
    args = [gr, gc, diff16, ea, wea, wrad, be1, we2, be2]
    if coord is None:
        return pl.pallas_call(
            _edge_body,
            grid=(e // _BE,),
            in_specs=in_specs,
            out_specs=edge_spec,
            out_shape=jax.ShapeDtypeStruct((e, d), F32),
        )(*args)
    wc1, bc1, wc2t = coord
    return pl.pallas_call(
        _edge_last_body,
        grid=(e // _BE,),
        in_specs=in_specs + [_full((d, d)), _full((1, d)), _full((1, d))],
        out_specs=[edge_spec, pl.BlockSpec((_BE, 128), lambda i: (i, 0))],
        out_shape=[
            jax.ShapeDtypeStruct((e, d), F32),
            jax.ShapeDtypeStruct((e, 128), F32),
        ],
    )(*args, wc1, bc1, wc2t)


def _node_body(h, agg, wn1h, wn1a, bn1, wn2, bn2, wrn, wcn,
               out_h, out_hr, out_hc):
    hv = h[...]
    t = _silu(_dot(hv, wn1h[...]) + _dot(agg[...], wn1a[...]) + bn1[...])
    hn = hv + _dot(t, wn2[...]) + bn2[...]
    out_h[...] = hn
    out_hr[...] = _dot(hn, wrn[...])
    out_hc[...] = _dot(hn, wcn[...])


def _node_mlp(h, agg, wn1h, wn1a, bn1, wn2, bn2, wrn, wcn):
    n, d = h.shape
    node_spec = pl.BlockSpec((_BN, d), lambda i: (i, 0))
    return pl.pallas_call(
        _node_body,
        grid=(n // _BN,),
        in_specs=[
            node_spec, node_spec,
            _full((d, d)), _full((d, d)), _full((1, d)),
            _full((d, d)), _full((1, d)),
            _full((d, d)), _full((d, d)),
        ],
        out_specs=[node_spec] * 3,
        out_shape=[jax.ShapeDtypeStruct((n, d), F32)] * 3,
    )(h, agg, wn1h, wn1a, bn1, wn2, bn2, wrn, wcn)


def _node_last_body(h, agg, x16, p0, p1, wn1h, wn1a, bn1, wn2, bn2,
                    out_h, out_x):
    hv = h[...]
    t = _silu(_dot(hv, wn1h[...]) + _dot(agg[...], wn1a[...]) + bn1[...])
    out_h[...] = hv + _dot(t, wn2[...]) + bn2[...]
    out_x[...] = x16[...] + p0[...] + p1[...]


def _node_mlp_last(h, agg, x16, p0, p1, wn1h, wn1a, bn1, wn2, bn2):
    n, d = h.shape
    node_spec = pl.BlockSpec((_BN, d), lambda i: (i, 0))
    nar_spec = pl.BlockSpec((_BN, 16), lambda i: (i, 0))
    return pl.pallas_call(
        _node_last_body,
        grid=(n // _BN,),
        in_specs=[
            node_spec, node_spec, nar_spec, nar_spec, nar_spec,
            _full((d, d)), _full((d, d)), _full((1, d)),
            _full((d, d)), _full((1, d)),
        ],
        out_specs=[node_spec, nar_spec],
        out_shape=[
            jax.ShapeDtypeStruct((n, d), F32),
            jax.ShapeDtypeStruct((n, 16), F32),
        ],
    )(h, agg, x16, p0, p1, wn1h, wn1a, bn1, wn2, bn2)


def kernel(h, x, edges, edge_attr, params):
    layers = params["layers"]
    n, d = h.shape
    e = edges.shape[1]
    de = edge_attr.shape[1]
    row = edges[0]
    col = edges[1]

    def split(p, with_coord):
        we1 = p["We1"]
        out = {
            "wr": we1[:d],
            "wc": we1[d:2 * d],
            "wrad": we1[2 * d:2 * d + 1],
            "wea": we1[2 * d + 1:],
            "be1": p["be1"].reshape(1, d),
            "we2": p["We2"],
            "be2": p["be2"].reshape(1, d),
            "wn1h": p["Wn1"][:d],
            "wn1a": p["Wn1"][d:],
            "bn1": p["bn1"].reshape(1, d),
            "wn2": p["Wn2"],
            "bn2": p["bn2"].reshape(1, d),
        }
        if with_coord:
            out["wc1"] = p["Wc1"]
            out["bc1"] = p["bc1"].reshape(1, d)
            out["wc2t"] = p["Wc2"].reshape(1, d)
        return out

    nl = len(layers)
    ps = [split(p, i == nl - 1) for i, p in enumerate(layers)]

    x16 = jnp.pad(x, ((0, 0), (0, 16 - x.shape[1])))
    x128 = jnp.pad(x, ((0, 0), (0, 128 - x.shape[1])))
    zeros128 = jnp.zeros((NPAD, 128), F32)

    gather128 = _make_gather2(n, 128, e)
    gather256 = _make_gather2(n, d, e)
    segsum = _make_segsum(e, d)
    segsum_part = _make_segsum_part(e)

    xr128, xc128 = gather128(x128, x128, row, col)
    diff16 = _coord_diff(xr128, xc128)

    hr, hc = _proj(h, ps[0]["wr"], ps[0]["wc"])
    for i, p in enumerate(ps):
        gr, gc = gather256(hr, hc, row, col)
        if i < nl - 1:
            m2 = _edge_mlp(gr, gc, diff16, edge_attr,
                           p["wea"], p["wrad"], p["be1"],
                           p["we2"], p["be2"])
            agg = segsum(m2, row, zeros128)
            h, hr, hc = _node_mlp(h, agg,
                                  p["wn1h"], p["wn1a"], p["bn1"],
                                  p["wn2"], p["bn2"],
                                  ps[i + 1]["wr"], ps[i + 1]["wc"])
        else:
            m2, trans = _edge_mlp(gr, gc, diff16, edge_attr,
                                  p["wea"], p["wrad"], p["be1"],
                                  p["we2"], p["be2"],
                                  coord=(p["wc1"], p["bc1"], p["wc2t"]))
            agg = segsum(m2, row, zeros128)
            parts = segsum_part(trans, row, zeros128)
            h, x16o = _node_mlp_last(h, agg, x16,
                                     parts[0, :n, :16], parts[1, :n, :16],
                                     p["wn1h"], p["wn1a"], p["bn1"],
                                     p["wn2"], p["bn2"])
    return (h, x16o[:, :3])


# speedup vs baseline: 3.3038x; 1.4837x over previous
import functools

import jax
import jax.numpy as jnp
from jax import lax
from jax.experimental import pallas as pl
from jax.experimental.pallas import tpu as pltpu
from jax.experimental.pallas import tpu_sc as plsc

F32 = jnp.float32

NC = 2
NS = 16
NW = NC * NS
NPAD = 10240
RPS = NPAD // NS


@functools.cache
def _mesh():
    return plsc.VectorSubcoreMesh(
        core_axis_name="c", subcore_axis_name="s",
        num_cores=NC, num_subcores=NS,
    )


@functools.cache
def _make_gather2sum(n_rows, width, n_edges, subtract):
    per = n_edges // NW
    assert per * NW == n_edges
    CH = 64
    n_full = per // CH
    tail = per - n_full * CH
    ngroups = n_full // 2
    assert n_full % 2 == 0 and tail % 8 == 0
    nslice = width // 16

    scratch = [
        pltpu.VMEM((per,), jnp.int32),
        pltpu.VMEM((per,), jnp.int32),
    ]
    for _ in range(2):
        scratch += [pltpu.VMEM((CH, width), F32)] * 3
    scratch += [pltpu.SemaphoreType.DMA] * 4
    if tail:
        scratch += [pltpu.VMEM((tail, width), F32)] * 3

    @functools.partial(
        pl.kernel,
        out_type=jax.ShapeDtypeStruct((n_edges, width), F32),
        mesh=_mesh(),
        scratch_types=scratch,
    )
    def gather2sum(ta, tb, ia, ib, out, idxa, idxb,
                   a0, b0, o0, a1, b1, o1, sa0, sb0, sa1, sb1, *tails):
        wid = lax.axis_index("s") * NC + lax.axis_index("c")
        base0 = wid * per
        pltpu.sync_copy(ia.at[pl.ds(base0, per)], idxa)
        pltpu.sync_copy(ib.at[pl.ds(base0, per)], idxb)
        pairs = ((a0, b0, o0, sa0, sb0), (a1, b1, o1, sa1, sb1))

        def issue(k, p):
            ba, bb, _, sa, sb = p
            pltpu.async_copy(ta.at[idxa.at[pl.ds(k * CH, CH)]], ba, sa)
            pltpu.async_copy(tb.at[idxb.at[pl.ds(k * CH, CH)]], bb, sb)

        def wait(k, p):
            ba, bb, _, sa, sb = p
            pltpu.make_async_copy(
                ta.at[idxa.at[pl.ds(k * CH, CH)]], ba, sa).wait()
            pltpu.make_async_copy(
                tb.at[idxb.at[pl.ds(k * CH, CH)]], bb, sb).wait()

        def combine(ba, bb, bo, n):
            def rowbody(r, c):
                for j in range(nslice):
                    sl = pl.ds(j * 16, 16)
                    if subtract:
                        bo[r, sl] = ba[r, sl] - bb[r, sl]
                    else:
                        bo[r, sl] = ba[r, sl] + bb[r, sl]
                return c
            lax.fori_loop(0, n, rowbody, 0)

        for b in range(2):
            issue(b, pairs[b])

        def group(g, c):
            for b in range(2):
                k = 2 * g + b
                p = pairs[b]
                wait(k, p)
                combine(p[0], p[1], p[2], CH)
                issue(k + 2, p)
                pltpu.sync_copy(p[2], out.at[pl.ds(base0 + k * CH, CH)])
            return c

        lax.fori_loop(0, ngroups - 1, group, 0)
        for b in range(2):
            k = 2 * (ngroups - 1) + b
            p = pairs[b]
            wait(k, p)
            combine(p[0], p[1], p[2], CH)
            pltpu.sync_copy(p[2], out.at[pl.ds(base0 + k * CH, CH)])
        if tail:
            tba, tbb, tbo = tails
            kb = n_full * CH
            pltpu.async_copy(
                ta.at[idxa.at[pl.ds(kb, tail)]], tba, sa0).wait()
            pltpu.async_copy(
                tb.at[idxb.at[pl.ds(kb, tail)]], tbb, sb0).wait()
            combine(tba, tbb, tbo, tail)
            pltpu.sync_copy(tbo, out.at[pl.ds(base0 + kb, tail)])

    return gather2sum


@functools.cache
def _make_segsum(n_edges, width):
    half = width // NC
    per = n_edges // NS
    assert per * NS == n_edges
    CH = 128
    n_full = per // CH
    tail = per - n_full * CH
    ngroups = n_full // 2
    assert n_full % 2 == 0 and tail % 8 == 0

    scratch = [
        pltpu.VMEM((CH,), jnp.int32),
        pltpu.VMEM((CH, half), F32),
        pltpu.VMEM((CH,), jnp.int32),
        pltpu.VMEM((CH, half), F32),
        pltpu.VMEM_SHARED((NPAD, half), F32),
        pltpu.SemaphoreType.DMA,
        pltpu.SemaphoreType.DMA,
        pltpu.SemaphoreType.DMA,
        pltpu.SemaphoreType.DMA,
    ]
    if tail:
        scratch += [
            pltpu.VMEM((tail,), jnp.int32),
            pltpu.VMEM((tail, half), F32),
        ]

    @functools.partial(
        pl.kernel,
        out_type=jax.ShapeDtypeStruct((NPAD, width), F32),
        mesh=_mesh(),
        scratch_types=scratch,
    )
    def segsum(vals, rows, zeros, out, i0, v0, i1, v1, acc,
               si0, sv0, si1, sv1, *tails):
        cid = lax.axis_index("c")
        sid = lax.axis_index("s")
        pltpu.sync_copy(
            zeros.at[pl.ds(sid * RPS, RPS), pl.ds(0, half)],
            acc.at[pl.ds(sid * RPS, RPS)],
        )
        plsc.subcore_barrier()

        base0 = sid * per
        pairs = ((i0, v0, si0, sv0), (i1, v1, si1, sv1))

        def issue(k, p):
            iv, vv, si, sv = p
            pltpu.async_copy(rows.at[pl.ds(base0 + k * CH, CH)], iv, si)
            pltpu.async_copy(
                vals.at[pl.ds(base0 + k * CH, CH), pl.ds(cid * half, half)],
                vv, sv)

        def wait(k, p):
            iv, vv, si, sv = p
            pltpu.make_async_copy(
                rows.at[pl.ds(base0 + k * CH, CH)], iv, si).wait()
            pltpu.make_async_copy(
                vals.at[pl.ds(base0 + k * CH, CH), pl.ds(cid * half, half)],
                vv, sv).wait()

        for b in range(2):
            issue(b, pairs[b])

        def group(g, c):
            for b in range(2):
                k = 2 * g + b
                p = pairs[b]
                wait(k, p)
                pltpu.sync_copy(p[1], acc.at[p[0]], add=True)
                issue(k + 2, p)
            return c

        lax.fori_loop(0, ngroups - 1, group, 0)
        for b in range(2):
            k = 2 * (ngroups - 1) + b
            p = pairs[b]
            wait(k, p)
            pltpu.sync_copy(p[1], acc.at[p[0]], add=True)
        if tail:
            ti, tv = tails
            kb = base0 + n_full * CH
            pltpu.sync_copy(rows.at[pl.ds(kb, tail)], ti)
            pltpu.sync_copy(
                vals.at[pl.ds(kb, tail), pl.ds(cid * half, half)], tv)
            pltpu.sync_copy(tv, acc.at[ti], add=True)

        plsc.subcore_barrier()
        pltpu.sync_copy(
            acc.at[pl.ds(sid * RPS, RPS)],
            out.at[pl.ds(sid * RPS, RPS), pl.ds(cid * half, half)],
        )

    return segsum


@functools.cache
def _make_segsum_part(n_edges):
    width = 128
    per_core = n_edges // NC
    per = per_core // NS
    CH = 64
    n_full = per // CH
    tail = per - n_full * CH
    ngroups = n_full // 2
    assert n_full % 2 == 0 and tail % 8 == 0

    scratch = [
        pltpu.VMEM((CH,), jnp.int32),
        pltpu.VMEM((CH, width), F32),
        pltpu.VMEM((CH,), jnp.int32),
        pltpu.VMEM((CH, width), F32),
        pltpu.VMEM_SHARED((NPAD, width), F32),
        pltpu.SemaphoreType.DMA,
        pltpu.SemaphoreType.DMA,
        pltpu.SemaphoreType.DMA,
        pltpu.SemaphoreType.DMA,
    ]
    if tail:
        scratch += [
            pltpu.VMEM((tail,), jnp.int32),
            pltpu.VMEM((tail, width), F32),
        ]

    @functools.partial(
        pl.kernel,
        out_type=jax.ShapeDtypeStruct((NC, NPAD, width), F32),
        mesh=_mesh(),
        scratch_types=scratch,
    )
    def segsum_part(vals, rows, zeros, out, i0, v0, i1, v1, acc,
                    si0, sv0, si1, sv1, *tails):
        cid = lax.axis_index("c")
        sid = lax.axis_index("s")
        pltpu.sync_copy(
            zeros.at[pl.ds(sid * RPS, RPS)],
            acc.at[pl.ds(sid * RPS, RPS)],
        )
        plsc.subcore_barrier()

        base0 = cid * per_core + sid * per
        pairs = ((i0, v0, si0, sv0), (i1, v1, si1, sv1))

        def issue(k, p):
            iv, vv, si, sv = p
            pltpu.async_copy(rows.at[pl.ds(base0 + k * CH, CH)], iv, si)
            pltpu.async_copy(vals.at[pl.ds(base0 + k * CH, CH)], vv, sv)

        def wait(k, p):
            iv, vv, si, sv = p
            pltpu.make_async_copy(
                rows.at[pl.ds(base0 + k * CH, CH)], iv, si).wait()
            pltpu.make_async_copy(
                vals.at[pl.ds(base0 + k * CH, CH)], vv, sv).wait()

        for b in range(2):
            issue(b, pairs[b])

        def group(g, c):
            for b in range(2):
                k = 2 * g + b
                p = pairs[b]
                wait(k, p)
                pltpu.sync_copy(p[1], acc.at[p[0]], add=True)
                issue(k + 2, p)
            return c

        lax.fori_loop(0, ngroups - 1, group, 0)
        for b in range(2):
            k = 2 * (ngroups - 1) + b
            p = pairs[b]
            wait(k, p)
            pltpu.sync_copy(p[1], acc.at[p[0]], add=True)
        if tail:
            ti, tv = tails
            kb = base0 + n_full * CH
            pltpu.sync_copy(rows.at[pl.ds(kb, tail)], ti)
            pltpu.sync_copy(vals.at[pl.ds(kb, tail)], tv)
            pltpu.sync_copy(tv, acc.at[ti], add=True)

        plsc.subcore_barrier()
        pltpu.sync_copy(
            acc.at[pl.ds(sid * RPS, RPS)],
            out.at[cid, pl.ds(sid * RPS, RPS)],
        )

    return segsum_part


def _silu(v):
    return v * jax.nn.sigmoid(v)


def _dot(a, b):
    return jnp.dot(a, b, preferred_element_type=F32)


_BN = 2000
_BE = 1600


def _full(shape):
    return pl.BlockSpec(shape, lambda i: (0,) * len(shape))


def _proj_body(h, wr, wc, hr, hc):
    hv = h[...]
    hr[...] = _dot(hv, wr[...])
    hc[...] = _dot(hv, wc[...])


def _proj(h, wr, wc):
    n, d = h.shape
    return pl.pallas_call(
        _proj_body,
        grid=(n // _BN,),
        in_specs=[
            pl.BlockSpec((_BN, d), lambda i: (i, 0)),
            _full((d, d)),
            _full((d, d)),
        ],
        out_specs=[pl.BlockSpec((_BN, d), lambda i: (i, 0))] * 2,
        out_shape=[jax.ShapeDtypeStruct((n, d), F32)] * 2,
    )(h, wr, wc)


def _edge_body(msum, diff, ea, wea, wrad, be1, we2, be2, out):
    d = diff[...][:, :16]
    radial = jnp.sum(d * d, axis=1, keepdims=True)
    pre = (
        msum[...] + _dot(ea[...], wea[...])
        + radial * wrad[...] + be1[...]
    )
    m = _silu(pre)
    out[...] = _silu(_dot(m, we2[...]) + be2[...])


def _edge_last_body(msum, diff, ea, wea, wrad, be1, we2, be2,
                    wc1, bc1, wc2t, out, trans):
    d = diff[...][:, :16]
    radial = jnp.sum(d * d, axis=1, keepdims=True)
    pre = (
        msum[...] + _dot(ea[...], wea[...])
        + radial * wrad[...] + be1[...]
    )
    m = _silu(pre)
    m2 = _silu(_dot(m, we2[...]) + be2[...])
    out[...] = m2
    c1 = _silu(_dot(m2, wc1[...]) + bc1[...])
    w = jnp.sum(c1 * wc2t[...], axis=1, keepdims=True)
    trans[...] = jnp.concatenate(
        [d * w, jnp.zeros((d.shape[0], 112), F32)], axis=1
    )


def _edge_mlp(msum, diff128, ea, wea, wrad, be1, we2, be2, coord=None):
    e, d = msum.shape
    de = ea.shape[1]
    edge_spec = pl.BlockSpec((_BE, d), lambda i: (i, 0))
    diff_spec = pl.BlockSpec((_BE, 128), lambda i: (i, 0))
    ea_spec = pl.BlockSpec((_BE, de), lambda i: (i, 0))
    in_specs = [
        edge_spec, diff_spec, ea_spec,
        _full((de, d)), _full((1, d)), _full((1, d)),
        _full((d, d)), _full((1, d)),
    ]
    args = [msum, diff128, ea, wea, wrad, be1, we2, be2]
    if coord is None:
        return pl.pallas_call(
            _edge_body,
            grid=(e // _BE,),
            in_specs=in_specs,
            out_specs=edge_spec,
            out_shape=jax.ShapeDtypeStruct((e, d), F32),
        )(*args)
    wc1, bc1, wc2t = coord
    return pl.pallas_call(
        _edge_last_body,
        grid=(e // _BE,),
        in_specs=in_specs + [_full((d, d)), _full((1, d)), _full((1, d))],
        out_specs=[edge_spec, pl.BlockSpec((_BE, 128), lambda i: (i, 0))],
        out_shape=[
            jax.ShapeDtypeStruct((e, d), F32),
            jax.ShapeDtypeStruct((e, 128), F32),
        ],
    )(*args, wc1, bc1, wc2t)


def _node_body(h, agg, wn1h, wn1a, bn1, wn2, bn2, wrn, wcn,
               out_h, out_hr, out_hc):
    hv = h[...]
    t = _silu(_dot(hv, wn1h[...]) + _dot(agg[...], wn1a[...]) + bn1[...])
    hn = hv + _dot(t, wn2[...]) + bn2[...]
    out_h[...] = hn
    out_hr[...] = _dot(hn, wrn[...])
    out_hc[...] = _dot(hn, wcn[...])


def _node_mlp(h, agg, wn1h, wn1a, bn1, wn2, bn2, wrn, wcn):
    n, d = h.shape
    node_spec = pl.BlockSpec((_BN, d), lambda i: (i, 0))
    return pl.pallas_call(
        _node_body,
        grid=(n // _BN,),
        in_specs=[
            node_spec, node_spec,
            _full((d, d)), _full((d, d)), _full((1, d)),
            _full((d, d)), _full((1, d)),
            _full((d, d)), _full((d, d)),
        ],
        out_specs=[node_spec] * 3,
        out_shape=[jax.ShapeDtypeStruct((n, d), F32)] * 3,
    )(h, agg, wn1h, wn1a, bn1, wn2, bn2, wrn, wcn)


def _node_last_body(h, agg, x16, p0, p1, wn1h, wn1a, bn1, wn2, bn2,
                    out_h, out_x):
    hv = h[...]
    t = _silu(_dot(hv, wn1h[...]) + _dot(agg[...], wn1a[...]) + bn1[...])
    out_h[...] = hv + _dot(t, wn2[...]) + bn2[...]
    out_x[...] = x16[...] + p0[...] + p1[...]


def _node_mlp_last(h, agg, x16, p0, p1, wn1h, wn1a, bn1, wn2, bn2):
    n, d = h.shape
    node_spec = pl.BlockSpec((_BN, d), lambda i: (i, 0))
    nar_spec = pl.BlockSpec((_BN, 16), lambda i: (i, 0))
    return pl.pallas_call(
        _node_last_body,
        grid=(n // _BN,),
        in_specs=[
            node_spec, node_spec, nar_spec, nar_spec, nar_spec,
            _full((d, d)), _full((d, d)), _full((1, d)),
            _full((d, d)), _full((1, d)),
        ],
        out_specs=[node_spec, nar_spec],
        out_shape=[
            jax.ShapeDtypeStruct((n, d), F32),
            jax.ShapeDtypeStruct((n, 16), F32),
        ],
    )(h, agg, x16, p0, p1, wn1h, wn1a, bn1, wn2, bn2)


def kernel(h, x, edges, edge_attr, params):
    layers = params["layers"]
    n, d = h.shape
    e = edges.shape[1]
    de = edge_attr.shape[1]
    row = edges[0]
    col = edges[1]

    def split(p, with_coord):
        we1 = p["We1"]
        out = {
            "wr": we1[:d],
            "wc": we1[d:2 * d],
            "wrad": we1[2 * d:2 * d + 1],
            "wea": we1[2 * d + 1:],
            "be1": p["be1"].reshape(1, d),
            "we2": p["We2"],
            "be2": p["be2"].reshape(1, d),
            "wn1h": p["Wn1"][:d],
            "wn1a": p["Wn1"][d:],
            "bn1": p["bn1"].reshape(1, d),
            "wn2": p["Wn2"],
            "bn2": p["bn2"].reshape(1, d),
        }
        if with_coord:
            out["wc1"] = p["Wc1"]
            out["bc1"] = p["bc1"].reshape(1, d)
            out["wc2t"] = p["Wc2"].reshape(1, d)
        return out

    nl = len(layers)
    ps = [split(p, i == nl - 1) for i, p in enumerate(layers)]

    x16 = jnp.pad(x, ((0, 0), (0, 16 - x.shape[1])))
    x128 = jnp.pad(x, ((0, 0), (0, 128 - x.shape[1])))
    zeros128 = jnp.zeros((NPAD, 128), F32)

    gdiff = _make_gather2sum(n, 128, e, True)
    gsum = _make_gather2sum(n, d, e, False)
    segsum = _make_segsum(e, d)
    segsum_part = _make_segsum_part(e)

    diff128 = gdiff(x128, x128, row, col)

    hr, hc = _proj(h, ps[0]["wr"], ps[0]["wc"])
    for i, p in enumerate(ps):
        msum = gsum(hr, hc, row, col)
        if i < nl - 1:
            m2 = _edge_mlp(msum, diff128, edge_attr,
                           p["wea"], p["wrad"], p["be1"],
                           p["we2"], p["be2"])
            agg = segsum(m2, row, zeros128)
            h, hr, hc = _node_mlp(h, agg,
                                  p["wn1h"], p["wn1a"], p["bn1"],
                                  p["wn2"], p["bn2"],
                                  ps[i + 1]["wr"], ps[i + 1]["wc"])
        else:
            m2, trans = _edge_mlp(msum, diff128, edge_attr,
                                  p["wea"], p["wrad"], p["be1"],
                                  p["we2"], p["be2"],
                                  coord=(p["wc1"], p["bc1"], p["wc2t"]))
            agg = segsum(m2, row, zeros128)
            parts = segsum_part(trans, row, zeros128)
            h, x16o = _node_mlp_last(h, agg, x16,
                                     parts[0, :n, :16], parts[1, :n, :16],
                                     p["wn1h"], p["wn1a"], p["bn1"],
                                     p["wn2"], p["bn2"])
    return (h, x16o[:, :3])
